# baseline (speedup 1.0000x reference)
import functools

import jax
import jax.numpy as jnp
from jax import lax
from jax.experimental import pallas as pl
from jax.experimental.pallas import tpu as pltpu
from jax.experimental.pallas import tpu_sc as plsc

D = 64
NC, NS = 2, 16
NW = NC * NS
SUB = 128
RB1 = 4096
CH_R = 128
L3 = 5
NSEG = 4


def _fuse_body(tab_a, feat_a, tab_b, feat_b, w1t_ref, w2t_ref, out_ref):
    w1t = w1t_ref[...]
    w2t = w2t_ref[...]
    dn = (((0,), (1,)), ((), ()))
    xa = lax.dot_general(tab_a[...], w1t, dn, preferred_element_type=jnp.float32)
    xa = xa + lax.dot_general(feat_a[...], w2t, dn,
                              preferred_element_type=jnp.float32)
    xb = lax.dot_general(tab_b[...], w1t, dn, preferred_element_type=jnp.float32)
    xb = xb + lax.dot_general(feat_b[...], w2t, dn,
                              preferred_element_type=jnp.float32)
    out_ref[...] = jnp.concatenate([xa, xb], axis=1)


def _gather_body(n_chunks, chunks_per_l, B, l0, ids_ref, fused_ref, out_ref,
                 idx0, idx1, rows0, rows1, gs0, gs1, ws0, ws1):
    wid = lax.axis_index("s") * NC + lax.axis_index("c")
    idx = (idx0, idx1)
    rows = (rows0, rows1)
    gsem = (gs0, gs1)
    wsem = (ws0, ws1)
    k_sub = CH_R // SUB
    chunk0 = wid * n_chunks

    def chunk_op(ci, s, drain):
        l_loc = ci // chunks_per_l
        rc = ci % chunks_per_l
        r0 = rc * CH_R
        l = l0 + l_loc
        row_a = (l * B + r0) // SUB
        row_b = (l * B + B // 2 + r0) // SUB
        orow0 = l_loc * (B // 2) + r0

        def _drain_prev():
            for c in range(2):
                pltpu.make_async_copy(
                    rows[s].at[c],
                    out_ref.at[pl.ds(orow0, CH_R), pl.ds(c * D, D)],
                    wsem[s]).wait()

        if drain is None:
            _drain_prev()
        else:
            pl.when(drain)(_drain_prev)

        pltpu.sync_copy(ids_ref.at[pl.ds(row_a, k_sub)],
                        idx[s].at[pl.ds(0, k_sub)])
        pltpu.sync_copy(ids_ref.at[pl.ds(row_b, k_sub)],
                        idx[s].at[pl.ds(k_sub, k_sub)])
        descs = []
        for c in range(2):
            for k in range(k_sub):
                descs.append(pltpu.async_copy(
                    fused_ref.at[idx[s].at[c * k_sub + k]],
                    rows[s].at[c, pl.ds(k * SUB, SUB)], gsem[s]))
        for dd in descs:
            dd.wait()
        for c in range(2):
            pltpu.async_copy(rows[s].at[c],
                             out_ref.at[pl.ds(orow0, CH_R), pl.ds(c * D, D)],
                             wsem[s])

    def step(g, carry):
        for s in range(2):
            chunk_op(chunk0 + 2 * g + s, s, g > 0)
        return carry

    lax.fori_loop(0, n_chunks // 2, step, 0)
    if n_chunks % 2:
        chunk_op(chunk0 + n_chunks - 1, 0, None)
    for s in range(2):
        for c in range(2):
            pltpu.make_async_copy(rows[s].at[c],
                                  out_ref.at[pl.ds(0, CH_R), pl.ds(c * D, D)],
                                  wsem[s]).wait()


def _pe_body(x_ref, pe_ref, o_ref):
    h = x_ref.shape[0] // L3
    for li in range(L3):
        x = x_ref[pl.ds(li * h, h), :]
        ya = x[:, :D].T
        yb = x[:, D:].T
        o_ref[li] = jnp.concatenate([ya, yb], axis=1) + pe_ref[li]


def _pe_body_alias(x_ref, pe_ref, prev_ref, o_ref):
    del prev_ref
    _pe_body(x_ref, pe_ref, o_ref)


def kernel(item_ids, item_table, features, W, b, pe):
    B, L = item_ids.shape
    n_rows = item_table.shape[0] - 1
    TOK = B * L
    chunks_per_l = B // (2 * CH_R)
    seg_l = L // NSEG
    n_chunks = (seg_l * chunks_per_l) // NW
    assert (seg_l * chunks_per_l) % NW == 0 and B % SUB == 0
    assert L % (NSEG * L3) == 0

    tab_t = item_table.T
    feat_t = features.T
    w_t = W.T
    w1_t = w_t[:, :D]
    w2_t = w_t[:, D:]

    n_grid = pl.cdiv(n_rows, 2 * RB1)
    pair = n_grid * RB1
    n_last = (n_rows + 1) // RB1
    b_map = lambda i: (0, jnp.minimum(i + n_grid, n_last))
    fused2 = pl.pallas_call(
        _fuse_body,
        grid=(n_grid,),
        in_specs=[
            pl.BlockSpec((D, RB1), lambda i: (0, i)),
            pl.BlockSpec((D, RB1), lambda i: (0, i)),
            pl.BlockSpec((D, RB1), b_map),
            pl.BlockSpec((D, RB1), b_map),
            pl.BlockSpec((D, D), lambda i: (0, 0)),
            pl.BlockSpec((D, D), lambda i: (0, 0)),
        ],
        out_specs=pl.BlockSpec((RB1, 2 * D), lambda i: (i, 0)),
        out_shape=jax.ShapeDtypeStruct((pair, 2 * D), jnp.float32),
    )(tab_t, feat_t, tab_t, feat_t, w1_t, w2_t)
    fused = fused2.reshape(2 * pair, D)

    ids_l = item_ids.T.astype(jnp.int32)
    idx_l = jnp.where(ids_l < pair, 2 * ids_l, 2 * ids_l - (2 * pair - 1))
    ids2 = idx_l.reshape(TOK // SUB, SUB)
    mesh = plsc.VectorSubcoreMesh(core_axis_name="c", subcore_axis_name="s")
    seg_rows = seg_l * (B // 2)
    gathered = []
    for seg in range(NSEG):
        gathered.append(pl.kernel(
            functools.partial(_gather_body, n_chunks, chunks_per_l, B,
                              seg * seg_l),
            out_type=jax.ShapeDtypeStruct((seg_rows, 2 * D), jnp.float32),
            mesh=mesh,
            compiler_params=pltpu.CompilerParams(use_tc_tiling_on_sc=False),
            scratch_types=[
                pltpu.VMEM((2 * (CH_R // SUB), SUB), jnp.int32),
                pltpu.VMEM((2 * (CH_R // SUB), SUB), jnp.int32),
                pltpu.VMEM((2, CH_R, D), jnp.float32),
                pltpu.VMEM((2, CH_R, D), jnp.float32),
                pltpu.SemaphoreType.DMA,
                pltpu.SemaphoreType.DMA,
                pltpu.SemaphoreType.DMA,
                pltpu.SemaphoreType.DMA,
            ],
        )(ids2, fused))

    pe_3 = (pe.reshape(L, D) + b[None, :])[:, :, None]
    seg_steps = seg_l // L3
    out_phys = None
    for seg in range(NSEG):
        off = seg * seg_steps
        in_specs = [
            pl.BlockSpec((L3 * (B // 2), 2 * D), lambda i: (i, 0)),
            pl.BlockSpec((L3, D, 1), lambda i, off=off: (i + off, 0, 0)),
        ]
        args = [gathered[seg], pe_3]
        body = _pe_body
        aliases = {}
        if seg > 0:
            in_specs.append(pl.BlockSpec(memory_space=pltpu.MemorySpace.HBM))
            args.append(out_phys)
            body = _pe_body_alias
            aliases = {2: 0}
        out_phys = pl.pallas_call(
            body,
            grid=(seg_steps,),
            in_specs=in_specs,
            out_specs=pl.BlockSpec((L3, D, B), lambda i, off=off: (i + off, 0, 0)),
            out_shape=jax.ShapeDtypeStruct((L, D, B), jnp.float32),
            input_output_aliases=aliases,
        )(*args)
    return jnp.transpose(out_phys, (2, 0, 1))


# speedup vs baseline: 1.1048x; 1.1048x over previous
import functools

import jax
import jax.numpy as jnp
from jax import lax
from jax.experimental import pallas as pl
from jax.experimental.pallas import tpu as pltpu
from jax.experimental.pallas import tpu_sc as plsc

D = 64
NC, NS = 2, 16
NW = NC * NS
SUB = 128
RB1 = 8192
CH_R = 256
L3 = 4
NSEG = 2


def _fuse_body(tab_a, feat_a, tab_b, feat_b, w1t_ref, w2t_ref, out_ref):
    w1t = w1t_ref[...]
    w2t = w2t_ref[...]
    dn = (((0,), (1,)), ((), ()))
    xa = lax.dot_general(tab_a[...], w1t, dn, preferred_element_type=jnp.float32)
    xa = xa + lax.dot_general(feat_a[...], w2t, dn,
                              preferred_element_type=jnp.float32)
    xb = lax.dot_general(tab_b[...], w1t, dn, preferred_element_type=jnp.float32)
    xb = xb + lax.dot_general(feat_b[...], w2t, dn,
                              preferred_element_type=jnp.float32)
    out_ref[...] = jnp.concatenate([xa, xb], axis=1)


def _gather_body(n_chunks, chunks_per_l, B, l0, ids_ref, fused_ref, out_ref,
                 idx0, idx1, rows0, rows1, gs0, gs1, ws0, ws1):
    wid = lax.axis_index("s") * NC + lax.axis_index("c")
    idx = (idx0, idx1)
    rows = (rows0, rows1)
    gsem = (gs0, gs1)
    wsem = (ws0, ws1)
    k_sub = CH_R // SUB
    chunk0 = wid * n_chunks

    def chunk_op(ci, s, drain):
        l_loc = ci // chunks_per_l
        rc = ci % chunks_per_l
        r0 = rc * CH_R
        l = l0 + l_loc
        row_a = (l * B + r0) // SUB
        row_b = (l * B + B // 2 + r0) // SUB
        orow0 = l_loc * (B // 2) + r0

        def _drain_prev():
            for c in range(2):
                pltpu.make_async_copy(
                    rows[s].at[c],
                    out_ref.at[pl.ds(orow0, CH_R), pl.ds(c * D, D)],
                    wsem[s]).wait()

        if drain is None:
            _drain_prev()
        else:
            pl.when(drain)(_drain_prev)

        pltpu.sync_copy(ids_ref.at[pl.ds(row_a, k_sub)],
                        idx[s].at[pl.ds(0, k_sub)])
        pltpu.sync_copy(ids_ref.at[pl.ds(row_b, k_sub)],
                        idx[s].at[pl.ds(k_sub, k_sub)])
        descs = []
        for c in range(2):
            for k in range(k_sub):
                descs.append(pltpu.async_copy(
                    fused_ref.at[idx[s].at[c * k_sub + k]],
                    rows[s].at[c, pl.ds(k * SUB, SUB)], gsem[s]))
        for dd in descs:
            dd.wait()
        for c in range(2):
            pltpu.async_copy(rows[s].at[c],
                             out_ref.at[pl.ds(orow0, CH_R), pl.ds(c * D, D)],
                             wsem[s])

    def step(g, carry):
        for s in range(2):
            chunk_op(chunk0 + 2 * g + s, s, g > 0)
        return carry

    lax.fori_loop(0, n_chunks // 2, step, 0)
    if n_chunks % 2:
        chunk_op(chunk0 + n_chunks - 1, 0, None)
    for s in range(2):
        for c in range(2):
            pltpu.make_async_copy(rows[s].at[c],
                                  out_ref.at[pl.ds(0, CH_R), pl.ds(c * D, D)],
                                  wsem[s]).wait()


def _pe_body(x_ref, pe_ref, o_ref):
    h = x_ref.shape[0] // L3
    for li in range(L3):
        x = x_ref[pl.ds(li * h, h), :]
        ya = x[:, :D].T
        yb = x[:, D:].T
        o_ref[li] = jnp.concatenate([ya, yb], axis=1) + pe_ref[li]


def _pe_body_alias(x_ref, pe_ref, prev_ref, o_ref):
    del prev_ref
    _pe_body(x_ref, pe_ref, o_ref)


def kernel(item_ids, item_table, features, W, b, pe):
    B, L = item_ids.shape
    n_rows = item_table.shape[0] - 1
    TOK = B * L
    chunks_per_l = B // (2 * CH_R)
    seg_l = L // NSEG
    n_chunks = (seg_l * chunks_per_l) // NW
    assert (seg_l * chunks_per_l) % NW == 0 and B % SUB == 0
    assert L % (NSEG * L3) == 0

    tab_t = item_table.T
    feat_t = features.T
    w_t = W.T
    w1_t = w_t[:, :D]
    w2_t = w_t[:, D:]

    n_grid = pl.cdiv(n_rows, 2 * RB1)
    pair = n_grid * RB1
    n_last = (n_rows + 1) // RB1
    b_map = lambda i: (0, jnp.minimum(i + n_grid, n_last))
    fused2 = pl.pallas_call(
        _fuse_body,
        grid=(n_grid,),
        in_specs=[
            pl.BlockSpec((D, RB1), lambda i: (0, i)),
            pl.BlockSpec((D, RB1), lambda i: (0, i)),
            pl.BlockSpec((D, RB1), b_map),
            pl.BlockSpec((D, RB1), b_map),
            pl.BlockSpec((D, D), lambda i: (0, 0)),
            pl.BlockSpec((D, D), lambda i: (0, 0)),
        ],
        out_specs=pl.BlockSpec((RB1, 2 * D), lambda i: (i, 0)),
        out_shape=jax.ShapeDtypeStruct((pair, 2 * D), jnp.float32),
    )(tab_t, feat_t, tab_t, feat_t, w1_t, w2_t)
    fused = fused2.reshape(2 * pair, D)

    ids_l = item_ids.T.astype(jnp.int32)
    idx_l = jnp.where(ids_l < pair, 2 * ids_l, 2 * ids_l - (2 * pair - 1))
    ids2 = idx_l.reshape(TOK // SUB, SUB)
    mesh = plsc.VectorSubcoreMesh(core_axis_name="c", subcore_axis_name="s")
    seg_rows = seg_l * (B // 2)
    gathered = []
    for seg in range(NSEG):
        gathered.append(pl.kernel(
            functools.partial(_gather_body, n_chunks, chunks_per_l, B,
                              seg * seg_l),
            out_type=jax.ShapeDtypeStruct((seg_rows, 2 * D), jnp.float32),
            mesh=mesh,
            compiler_params=pltpu.CompilerParams(use_tc_tiling_on_sc=False),
            scratch_types=[
                pltpu.VMEM((2 * (CH_R // SUB), SUB), jnp.int32),
                pltpu.VMEM((2 * (CH_R // SUB), SUB), jnp.int32),
                pltpu.VMEM((2, CH_R, D), jnp.float32),
                pltpu.VMEM((2, CH_R, D), jnp.float32),
                pltpu.SemaphoreType.DMA,
                pltpu.SemaphoreType.DMA,
                pltpu.SemaphoreType.DMA,
                pltpu.SemaphoreType.DMA,
            ],
        )(ids2, fused))

    pe_3 = (pe.reshape(L, D) + b[None, :])[:, :, None]
    seg_steps = seg_l // L3
    out_phys = None
    for seg in range(NSEG):
        off = seg * seg_steps
        in_specs = [
            pl.BlockSpec((L3 * (B // 2), 2 * D), lambda i: (i, 0)),
            pl.BlockSpec((L3, D, 1), lambda i, off=off: (i + off, 0, 0)),
        ]
        args = [gathered[seg], pe_3]
        body = _pe_body
        aliases = {}
        if seg > 0:
            in_specs.append(pl.BlockSpec(memory_space=pltpu.MemorySpace.HBM))
            args.append(out_phys)
            body = _pe_body_alias
            aliases = {2: 0}
        out_phys = pl.pallas_call(
            body,
            grid=(seg_steps,),
            in_specs=in_specs,
            out_specs=pl.BlockSpec((L3, D, B), lambda i, off=off: (i + off, 0, 0)),
            out_shape=jax.ShapeDtypeStruct((L, D, B), jnp.float32),
            input_output_aliases=aliases,
        )(*args)
    return jnp.transpose(out_phys, (2, 0, 1))


# speedup vs baseline: 1.1054x; 1.0005x over previous
import functools

import jax
import jax.numpy as jnp
from jax import lax
from jax.experimental import pallas as pl
from jax.experimental.pallas import tpu as pltpu
from jax.experimental.pallas import tpu_sc as plsc

D = 64
NC, NS = 2, 16
NW = NC * NS
SUB = 128
RB1 = 8192
CH_R = 256
L3 = 5
NSEG = 2


def _fuse_body(tab_a, feat_a, tab_b, feat_b, w1t_ref, w2t_ref, out_ref):
    w1t = w1t_ref[...]
    w2t = w2t_ref[...]
    dn = (((0,), (1,)), ((), ()))
    xa = lax.dot_general(tab_a[...], w1t, dn, preferred_element_type=jnp.float32)
    xa = xa + lax.dot_general(feat_a[...], w2t, dn,
                              preferred_element_type=jnp.float32)
    xb = lax.dot_general(tab_b[...], w1t, dn, preferred_element_type=jnp.float32)
    xb = xb + lax.dot_general(feat_b[...], w2t, dn,
                              preferred_element_type=jnp.float32)
    out_ref[...] = jnp.concatenate([xa, xb], axis=1)


def _gather_body(n_chunks, chunks_per_l, B, l0, ids_ref, fused_ref, out_ref,
                 idx0, idx1, rows0, rows1, gs0, gs1, ws0, ws1):
    wid = lax.axis_index("s") * NC + lax.axis_index("c")
    idx = (idx0, idx1)
    rows = (rows0, rows1)
    gsem = (gs0, gs1)
    wsem = (ws0, ws1)
    k_sub = CH_R // SUB
    chunk0 = wid * n_chunks

    def chunk_op(ci, s, drain):
        l_loc = ci // chunks_per_l
        rc = ci % chunks_per_l
        r0 = rc * CH_R
        l = l0 + l_loc
        row_a = (l * B + r0) // SUB
        row_b = (l * B + B // 2 + r0) // SUB
        orow0 = l_loc * (B // 2) + r0

        def _drain_prev():
            for c in range(2):
                pltpu.make_async_copy(
                    rows[s].at[c],
                    out_ref.at[pl.ds(orow0, CH_R), pl.ds(c * D, D)],
                    wsem[s]).wait()

        if drain is None:
            _drain_prev()
        else:
            pl.when(drain)(_drain_prev)

        pltpu.sync_copy(ids_ref.at[pl.ds(row_a, k_sub)],
                        idx[s].at[pl.ds(0, k_sub)])
        pltpu.sync_copy(ids_ref.at[pl.ds(row_b, k_sub)],
                        idx[s].at[pl.ds(k_sub, k_sub)])
        descs = []
        for c in range(2):
            for k in range(k_sub):
                descs.append(pltpu.async_copy(
                    fused_ref.at[idx[s].at[c * k_sub + k]],
                    rows[s].at[c, pl.ds(k * SUB, SUB)], gsem[s]))
        for dd in descs:
            dd.wait()
        for c in range(2):
            pltpu.async_copy(rows[s].at[c],
                             out_ref.at[pl.ds(orow0, CH_R), pl.ds(c * D, D)],
                             wsem[s])

    def step(g, carry):
        for s in range(2):
            chunk_op(chunk0 + 2 * g + s, s, g > 0)
        return carry

    lax.fori_loop(0, n_chunks // 2, step, 0)
    if n_chunks % 2:
        chunk_op(chunk0 + n_chunks - 1, 0, None)
    for s in range(2):
        for c in range(2):
            pltpu.make_async_copy(rows[s].at[c],
                                  out_ref.at[pl.ds(0, CH_R), pl.ds(c * D, D)],
                                  wsem[s]).wait()


def _pe_body(x_ref, pe_ref, o_ref):
    h = x_ref.shape[0] // L3
    for li in range(L3):
        x = x_ref[pl.ds(li * h, h), :]
        ya = x[:, :D].T
        yb = x[:, D:].T
        o_ref[li] = jnp.concatenate([ya, yb], axis=1) + pe_ref[li]


def _pe_body_alias(x_ref, pe_ref, prev_ref, o_ref):
    del prev_ref
    _pe_body(x_ref, pe_ref, o_ref)


def kernel(item_ids, item_table, features, W, b, pe):
    B, L = item_ids.shape
    n_rows = item_table.shape[0] - 1
    TOK = B * L
    chunks_per_l = B // (2 * CH_R)
    seg_l = L // NSEG
    n_chunks = (seg_l * chunks_per_l) // NW
    assert (seg_l * chunks_per_l) % NW == 0 and B % SUB == 0
    assert L % (NSEG * L3) == 0

    tab_t = item_table.T
    feat_t = features.T
    w_t = W.T
    w1_t = w_t[:, :D]
    w2_t = w_t[:, D:]

    n_grid = pl.cdiv(n_rows, 2 * RB1)
    pair = n_grid * RB1
    n_last = (n_rows + 1) // RB1
    b_map = lambda i: (0, jnp.minimum(i + n_grid, n_last))
    fused2 = pl.pallas_call(
        _fuse_body,
        grid=(n_grid,),
        in_specs=[
            pl.BlockSpec((D, RB1), lambda i: (0, i)),
            pl.BlockSpec((D, RB1), lambda i: (0, i)),
            pl.BlockSpec((D, RB1), b_map),
            pl.BlockSpec((D, RB1), b_map),
            pl.BlockSpec((D, D), lambda i: (0, 0)),
            pl.BlockSpec((D, D), lambda i: (0, 0)),
        ],
        out_specs=pl.BlockSpec((RB1, 2 * D), lambda i: (i, 0)),
        out_shape=jax.ShapeDtypeStruct((pair, 2 * D), jnp.float32),
    )(tab_t, feat_t, tab_t, feat_t, w1_t, w2_t)
    fused = fused2.reshape(2 * pair, D)

    ids_l = item_ids.T.astype(jnp.int32)
    idx_l = jnp.where(ids_l < pair, 2 * ids_l, 2 * ids_l - (2 * pair - 1))
    ids2 = idx_l.reshape(TOK // SUB, SUB)
    mesh = plsc.VectorSubcoreMesh(core_axis_name="c", subcore_axis_name="s")
    seg_rows = seg_l * (B // 2)
    gathered = []
    for seg in range(NSEG):
        gathered.append(pl.kernel(
            functools.partial(_gather_body, n_chunks, chunks_per_l, B,
                              seg * seg_l),
            out_type=jax.ShapeDtypeStruct((seg_rows, 2 * D), jnp.float32),
            mesh=mesh,
            compiler_params=pltpu.CompilerParams(use_tc_tiling_on_sc=False),
            scratch_types=[
                pltpu.VMEM((2 * (CH_R // SUB), SUB), jnp.int32),
                pltpu.VMEM((2 * (CH_R // SUB), SUB), jnp.int32),
                pltpu.VMEM((2, CH_R, D), jnp.float32),
                pltpu.VMEM((2, CH_R, D), jnp.float32),
                pltpu.SemaphoreType.DMA,
                pltpu.SemaphoreType.DMA,
                pltpu.SemaphoreType.DMA,
                pltpu.SemaphoreType.DMA,
            ],
        )(ids2, fused))

    pe_3 = (pe.reshape(L, D) + b[None, :])[:, :, None]
    seg_steps = seg_l // L3
    out_phys = None
    for seg in range(NSEG):
        off = seg * seg_steps
        in_specs = [
            pl.BlockSpec((L3 * (B // 2), 2 * D), lambda i: (i, 0)),
            pl.BlockSpec((L3, D, 1), lambda i, off=off: (i + off, 0, 0)),
        ]
        args = [gathered[seg], pe_3]
        body = _pe_body
        aliases = {}
        if seg > 0:
            in_specs.append(pl.BlockSpec(memory_space=pltpu.MemorySpace.HBM))
            args.append(out_phys)
            body = _pe_body_alias
            aliases = {2: 0}
        out_phys = pl.pallas_call(
            body,
            grid=(seg_steps,),
            in_specs=in_specs,
            out_specs=pl.BlockSpec((L3, D, B), lambda i, off=off: (i + off, 0, 0)),
            out_shape=jax.ShapeDtypeStruct((L, D, B), jnp.float32),
            input_output_aliases=aliases,
        )(*args)
    return jnp.transpose(out_phys, (2, 0, 1))
